# Initial kernel scaffold; baseline (speedup 1.0000x reference)
import jax
import jax.numpy as jnp
from jax.experimental import pallas as pl


def kernel(X, edge_index, weights, bias):
    raise NotImplementedError("write your pallas kernel here")


# speedup vs baseline: 7.0002x; 7.0002x over previous
import functools

import jax
import jax.numpy as jnp
from jax import lax
from jax.experimental import pallas as pl
from jax.experimental.pallas import tpu as pltpu
from jax.experimental.pallas import tpu_sc as plsc

N_NODES = 10000
N_EDGES = 320000
D_IN = 128
D_OUT = 128

NC = 2
NS = 16
NW = NC * NS
EW = N_EDGES // NW
CHUNK = 80
NCHUNK = EW // CHUNK
STRIPE = 1000
DEG_W = 16

_mesh = lambda: plsc.VectorSubcoreMesh(core_axis_name="c", subcore_axis_name="s")


@functools.partial(
    pl.kernel,
    out_type=jax.ShapeDtypeStruct((NC, N_NODES, DEG_W), jnp.float32),
    mesh=_mesh(),
    scratch_types=[
        pltpu.VMEM((NCHUNK, CHUNK), jnp.int32),
        pltpu.VMEM((CHUNK, DEG_W), jnp.float32),
        pltpu.VMEM_SHARED((N_NODES, DEG_W), jnp.float32),
    ],
)
def _deg_kernel(row2d_hbm, ones_hbm, zeros_hbm, out_hbm, rowbuf, ones_v, acc):
    cid = lax.axis_index("c")
    sid = lax.axis_index("s")
    wid = cid * NS + sid

    @pl.when(sid < N_NODES // STRIPE)
    def _():
        pltpu.sync_copy(zeros_hbm, acc.at[pl.ds(sid * STRIPE, STRIPE), :])
    pltpu.sync_copy(ones_hbm, ones_v)
    pltpu.sync_copy(row2d_hbm.at[wid], rowbuf)
    plsc.subcore_barrier()

    def body(c, _):
        pltpu.sync_copy(ones_v, acc.at[rowbuf.at[c]], add=True)
        return 0
    lax.fori_loop(0, NCHUNK, body, 0)

    plsc.subcore_barrier()
    @pl.when(sid < N_NODES // STRIPE)
    def _():
        pltpu.sync_copy(acc.at[pl.ds(sid * STRIPE, STRIPE), :],
                        out_hbm.at[cid, pl.ds(sid * STRIPE, STRIPE), :])


@functools.partial(
    pl.kernel,
    out_type=jax.ShapeDtypeStruct((NC, N_NODES, D_OUT), jnp.float32),
    mesh=_mesh(),
    scratch_types=[
        pltpu.VMEM((NCHUNK, CHUNK), jnp.int32),
        pltpu.VMEM((EW,), jnp.int32),
        pltpu.VMEM((CHUNK, D_OUT), jnp.float32),
        pltpu.VMEM_SHARED((N_NODES, D_OUT), jnp.float32),
        pltpu.SemaphoreType.DMA,
    ],
)
def _scatter_kernel(m_hbm, row2d_hbm, col_hbm, zeros_hbm, out_hbm,
                    rowbuf, colbuf, gbuf, acc, sem):
    cid = lax.axis_index("c")
    sid = lax.axis_index("s")
    wid = cid * NS + sid
    base = wid * EW

    @pl.when(sid < N_NODES // STRIPE)
    def _():
        pltpu.sync_copy(zeros_hbm, acc.at[pl.ds(sid * STRIPE, STRIPE), :])
    pltpu.sync_copy(row2d_hbm.at[wid], rowbuf)
    pltpu.sync_copy(col_hbm.at[pl.ds(base, EW)], colbuf)
    plsc.subcore_barrier()

    def body(c, _):
        pltpu.async_copy(m_hbm.at[colbuf.at[pl.ds(c * CHUNK, CHUNK)]],
                         gbuf, sem).wait()
        pltpu.sync_copy(gbuf, acc.at[rowbuf.at[c]], add=True)
        return 0
    lax.fori_loop(0, NCHUNK, body, 0)

    plsc.subcore_barrier()
    @pl.when(sid < N_NODES // STRIPE)
    def _():
        pltpu.sync_copy(acc.at[pl.ds(sid * STRIPE, STRIPE), :],
                        out_hbm.at[cid, pl.ds(sid * STRIPE, STRIPE), :])


def _mm_body(x_ref, w_ref, deg_ref, o_ref):
    deg = deg_ref[0, :, 0] + deg_ref[1, :, 0]
    d = lax.rsqrt(deg + 1.0)
    y = jnp.dot(x_ref[...], w_ref[...], preferred_element_type=jnp.float32)
    o_ref[...] = y * d[:, None]


def _fin_body(am_ref, deg_ref, b_ref, o_ref):
    deg = deg_ref[0, :, 0] + deg_ref[1, :, 0]
    d = lax.rsqrt(deg + 1.0)
    s = am_ref[0] + am_ref[1]
    o_ref[...] = s * d[:, None] + b_ref[0, :]


@jax.jit
def kernel(X, edge_index, weights, bias):
    row = edge_index[0].astype(jnp.int32)
    col = edge_index[1].astype(jnp.int32)
    row2d = row.reshape(NW, NCHUNK, CHUNK)
    ones = jnp.ones((CHUNK, DEG_W), jnp.float32)
    zeros = jnp.zeros((STRIPE, D_OUT), jnp.float32)
    zeros_deg = jnp.zeros((STRIPE, DEG_W), jnp.float32)

    deg2 = _deg_kernel(row2d, ones, zeros_deg)

    m = pl.pallas_call(
        _mm_body,
        out_shape=jax.ShapeDtypeStruct((N_NODES, D_OUT), jnp.float32),
    )(X, weights, deg2)

    am2 = _scatter_kernel(m, row2d, col, zeros)

    out = pl.pallas_call(
        _fin_body,
        out_shape=jax.ShapeDtypeStruct((N_NODES, D_OUT), jnp.float32),
    )(am2, deg2, bias.reshape(1, D_OUT))
    return out


# speedup vs baseline: 8.1579x; 1.1654x over previous
import functools

import jax
import jax.numpy as jnp
from jax import lax
from jax.experimental import pallas as pl
from jax.experimental.pallas import tpu as pltpu
from jax.experimental.pallas import tpu_sc as plsc

N_NODES = 10000
N_EDGES = 320000
D_IN = 128
D_OUT = 128

NC = 2
NS = 16
NW = NC * NS
EW = N_EDGES // NW
CHUNK = 80
NCHUNK = EW // CHUNK
UNROLL = 2
KBLK = 5
DCHUNK = 80
NDCHUNK = EW // DCHUNK
STRIPE = 1000
DEG_W = 16

_mesh = lambda: plsc.VectorSubcoreMesh(core_axis_name="c", subcore_axis_name="s")


@functools.partial(
    pl.kernel,
    out_type=jax.ShapeDtypeStruct((NC, N_NODES, DEG_W), jnp.float32),
    mesh=_mesh(),
    scratch_types=[
        pltpu.VMEM((NDCHUNK, DCHUNK), jnp.int32),
        pltpu.VMEM((DCHUNK, DEG_W), jnp.float32),
        pltpu.VMEM_SHARED((N_NODES, DEG_W), jnp.float32),
    ],
)
def _deg_kernel(row2d_hbm, ones_hbm, zeros_hbm, out_hbm, rowbuf, ones_v, acc):
    cid = lax.axis_index("c")
    sid = lax.axis_index("s")
    wid = cid * NS + sid

    @pl.when(sid < N_NODES // STRIPE)
    def _():
        pltpu.sync_copy(zeros_hbm, acc.at[pl.ds(sid * STRIPE, STRIPE), :])
    pltpu.sync_copy(ones_hbm, ones_v)
    pltpu.sync_copy(row2d_hbm.at[wid], rowbuf)
    plsc.subcore_barrier()

    def body(c, _):
        pltpu.sync_copy(ones_v, acc.at[rowbuf.at[c]], add=True)
        return 0
    lax.fori_loop(0, NDCHUNK, body, 0)

    plsc.subcore_barrier()
    @pl.when(sid < N_NODES // STRIPE)
    def _():
        pltpu.sync_copy(acc.at[pl.ds(sid * STRIPE, STRIPE), :],
                        out_hbm.at[cid, pl.ds(sid * STRIPE, STRIPE), :])


@functools.partial(
    pl.kernel,
    out_type=jax.ShapeDtypeStruct((NC, N_NODES, D_OUT), jnp.float32),
    mesh=_mesh(),
    scratch_types=[
        pltpu.VMEM((NCHUNK, CHUNK), jnp.int32),
        pltpu.VMEM((EW,), jnp.int32),
        pltpu.VMEM((UNROLL, CHUNK, D_OUT), jnp.float32),
        pltpu.VMEM_SHARED((N_NODES, D_OUT), jnp.float32),
        pltpu.SemaphoreType.DMA((UNROLL,)),
    ],
)
def _scatter_kernel(m_hbm, row2d_hbm, col_hbm, zeros_hbm, out_hbm,
                    rowbuf, colbuf, gbuf, acc, sem):
    cid = lax.axis_index("c")
    sid = lax.axis_index("s")
    wid = cid * NS + sid
    base = wid * EW

    @pl.when(sid < N_NODES // STRIPE)
    def _():
        pltpu.sync_copy(zeros_hbm, acc.at[pl.ds(sid * STRIPE, STRIPE), :])
    pltpu.sync_copy(row2d_hbm.at[wid], rowbuf)
    pltpu.sync_copy(col_hbm.at[pl.ds(base, EW)], colbuf)
    plsc.subcore_barrier()

    def _gather(c, b):
        return pltpu.async_copy(
            m_hbm.at[colbuf.at[pl.ds(c * CHUNK, CHUNK)]], gbuf.at[b],
            sem.at[b])

    def body(i, _):
        c0 = i * KBLK
        d = _gather(c0, 0)
        for j in range(KBLK):
            d.wait()
            if j + 1 < KBLK:
                d = _gather(c0 + j + 1, (j + 1) % 2)
            pltpu.sync_copy(gbuf.at[j % 2], acc.at[rowbuf.at[c0 + j]], add=True)
        return 0
    lax.fori_loop(0, NCHUNK // KBLK, body, 0)

    plsc.subcore_barrier()
    @pl.when(sid < N_NODES // STRIPE)
    def _():
        pltpu.sync_copy(acc.at[pl.ds(sid * STRIPE, STRIPE), :],
                        out_hbm.at[cid, pl.ds(sid * STRIPE, STRIPE), :])


def _mm_body(x_ref, w_ref, deg_ref, o_ref):
    deg = deg_ref[0, :, 0] + deg_ref[1, :, 0]
    d = lax.rsqrt(deg + 1.0)
    y = jnp.dot(x_ref[...], w_ref[...], preferred_element_type=jnp.float32)
    o_ref[...] = y * d[:, None]


def _fin_body(am_ref, deg_ref, b_ref, o_ref):
    deg = deg_ref[0, :, 0] + deg_ref[1, :, 0]
    d = lax.rsqrt(deg + 1.0)
    s = am_ref[0] + am_ref[1]
    o_ref[...] = s * d[:, None] + b_ref[0, :]


@jax.jit
def kernel(X, edge_index, weights, bias):
    row = edge_index[0].astype(jnp.int32)
    col = edge_index[1].astype(jnp.int32)
    row2d = row.reshape(NW, NCHUNK, CHUNK)
    ones = jnp.ones((DCHUNK, DEG_W), jnp.float32)
    zeros = jnp.zeros((STRIPE, D_OUT), jnp.float32)
    zeros_deg = jnp.zeros((STRIPE, DEG_W), jnp.float32)

    deg2 = _deg_kernel(row2d, ones, zeros_deg)

    m = pl.pallas_call(
        _mm_body,
        out_shape=jax.ShapeDtypeStruct((N_NODES, D_OUT), jnp.float32),
    )(X, weights, deg2)

    am2 = _scatter_kernel(m, row2d, col, zeros)

    out = pl.pallas_call(
        _fin_body,
        out_shape=jax.ShapeDtypeStruct((N_NODES, D_OUT), jnp.float32),
    )(am2, deg2, bias.reshape(1, D_OUT))
    return out
